# Initial kernel scaffold; baseline (speedup 1.0000x reference)
import functools

import jax
import jax.numpy as jnp
from jax import lax
from jax.experimental import pallas as pl
from jax.experimental.pallas import tpu as pltpu
from jax.experimental.pallas import tpu_sc as plsc

VOCAB = 1000000
EMB = 64
HID = 256
OUT = 100
B = 16384
L = 50

NUM_CORES = 2
NUM_SUBCORES = 16
NW = NUM_CORES * NUM_SUBCORES
BPW = B // NW
ELEMS_PER_CHUNK = 2
ROWS_PER_CHUNK = ELEMS_PER_CHUNK * L
CHUNKS = BPW // ELEMS_PER_CHUNK
NVREG = EMB // 16


def _pool_body(idx_hbm, table_hbm, out_hbm, idx_v, rows0, rows1, pooled_v,
               sem0, sem1):
    wid = lax.axis_index("s") * NUM_CORES + lax.axis_index("c")

    pltpu.sync_copy(idx_hbm.at[wid], idx_v)

    bufs = (rows0, rows1)
    sems = (sem0, sem1)

    pltpu.async_copy(table_hbm.at[idx_v.at[0]], rows0, sem0)
    pltpu.async_copy(table_hbm.at[idx_v.at[1]], rows1, sem1)

    inv_l = jnp.float32(1.0 / L)

    def outer(j, carry):
        for b in range(2):
            i = j * 2 + b
            buf = bufs[b]
            sem = sems[b]
            pltpu.make_async_copy(table_hbm.at[idx_v.at[i]], buf, sem).wait()
            for s in range(ELEMS_PER_CHUNK):
                accs = [buf[s * L, pl.ds(16 * k, 16)] for k in range(NVREG)]
                for r in range(1, L):
                    for k in range(NVREG):
                        accs[k] = accs[k] + buf[s * L + r, pl.ds(16 * k, 16)]
                row = i * ELEMS_PER_CHUNK + s
                for k in range(NVREG):
                    pooled_v[row, pl.ds(16 * k, 16)] = accs[k] * inv_l
            @pl.when(j < CHUNKS // 2 - 1)
            def _():
                pltpu.async_copy(table_hbm.at[idx_v.at[i + 2]], buf, sem)
        return carry

    lax.fori_loop(0, CHUNKS // 2, outer, 0, unroll=False)

    pltpu.sync_copy(pooled_v, out_hbm.at[pl.ds(wid * BPW, BPW)])


@jax.jit
def _gather_pool(idx3, emb_table):
    mesh = plsc.VectorSubcoreMesh(core_axis_name="c", subcore_axis_name="s")
    return pl.kernel(
        _pool_body,
        out_type=jax.ShapeDtypeStruct((B, EMB), jnp.float32),
        mesh=mesh,
        scratch_types=[
            pltpu.VMEM((CHUNKS, ROWS_PER_CHUNK), jnp.int32),
            pltpu.VMEM((ROWS_PER_CHUNK, EMB), jnp.float32),
            pltpu.VMEM((ROWS_PER_CHUNK, EMB), jnp.float32),
            pltpu.VMEM((BPW, EMB), jnp.float32),
            pltpu.SemaphoreType.DMA,
            pltpu.SemaphoreType.DMA,
        ],
    )(idx3, emb_table)


def _mlp_body(x_ref, w1_ref, b1_ref, w2_ref, b2_ref, o_ref):
    x = x_ref[...]
    h = jnp.dot(x, w1_ref[...], preferred_element_type=jnp.float32)
    h = jnp.maximum(h + b1_ref[...], 0.0)
    logits = jnp.dot(h, w2_ref[...], preferred_element_type=jnp.float32)
    logits = logits + b2_ref[...]
    m = jnp.max(logits, axis=-1, keepdims=True)
    e = jnp.exp(logits - m)
    lse = jnp.log(jnp.sum(e, axis=-1, keepdims=True)) + m
    o_ref[...] = logits - lse


def _mlp(pooled, W1, b1, W2, b2):
    BM = 2048
    grid = (B // BM,)
    return pl.pallas_call(
        _mlp_body,
        grid=grid,
        in_specs=[
            pl.BlockSpec((BM, EMB), lambda i: (i, 0)),
            pl.BlockSpec((EMB, HID), lambda i: (0, 0)),
            pl.BlockSpec((1, HID), lambda i: (0, 0)),
            pl.BlockSpec((HID, OUT), lambda i: (0, 0)),
            pl.BlockSpec((1, OUT), lambda i: (0, 0)),
        ],
        out_specs=pl.BlockSpec((BM, OUT), lambda i: (i, 0)),
        out_shape=jax.ShapeDtypeStruct((B, OUT), jnp.float32),
    )(pooled, W1, b1.reshape(1, HID), W2, b2.reshape(1, OUT))


def kernel(input, emb_table, W1, b1, W2, b2):
    idx3 = input.astype(jnp.int32).reshape(NW, CHUNKS, ROWS_PER_CHUNK)
    pooled = _gather_pool(idx3, emb_table)
    return _mlp(pooled, W1, b1, W2, b2)


# speedup vs baseline: 2.5166x; 2.5166x over previous
import functools

import jax
import jax.numpy as jnp
from jax import lax
from jax.experimental import pallas as pl
from jax.experimental.pallas import tpu as pltpu
from jax.experimental.pallas import tpu_sc as plsc

VOCAB = 1000000
EMB = 64
HID = 256
OUT = 100
B = 16384
L = 50

NUM_CORES = 2
NUM_SUBCORES = 16
NW = NUM_CORES * NUM_SUBCORES
BPW = B // NW
ELEMS_PER_CHUNK = 2
ROWS_PER_CHUNK = ELEMS_PER_CHUNK * L
CHUNKS = BPW // ELEMS_PER_CHUNK
NVREG = EMB // 16


def _pool_body(idx_hbm, table_hbm, out_hbm, idx_v, rows0, rows1, pooled_v,
               sem0, sem1):
    wid = lax.axis_index("s") * NUM_CORES + lax.axis_index("c")

    pltpu.sync_copy(idx_hbm.at[wid], idx_v)

    bufs = (rows0, rows1)
    sems = (sem0, sem1)

    pltpu.async_copy(table_hbm.at[idx_v.at[0]], rows0, sem0)
    pltpu.async_copy(table_hbm.at[idx_v.at[1]], rows1, sem1)

    inv_l = jnp.float32(1.0 / L)

    def outer(j, carry):
        for b in range(2):
            i = j * 2 + b
            buf = bufs[b]
            sem = sems[b]
            pltpu.make_async_copy(table_hbm.at[idx_v.at[i]], buf, sem).wait()
            for s in range(ELEMS_PER_CHUNK):
                accs = [buf[s * L, pl.ds(16 * k, 16)] for k in range(NVREG)]
                for r in range(1, L):
                    for k in range(NVREG):
                        accs[k] = accs[k] + buf[s * L + r, pl.ds(16 * k, 16)]
                row = i * ELEMS_PER_CHUNK + s
                for k in range(NVREG):
                    pooled_v[row, pl.ds(16 * k, 16)] = accs[k] * inv_l
            @pl.when(j < CHUNKS // 2 - 1)
            def _():
                pltpu.async_copy(table_hbm.at[idx_v.at[i + 2]], buf, sem)
        return carry

    lax.fori_loop(0, CHUNKS // 2, outer, 0, unroll=False)

    pltpu.sync_copy(pooled_v, out_hbm.at[pl.ds(wid * BPW, BPW)])


@jax.jit
def _gather_pool(idx3, emb_table):
    mesh = plsc.VectorSubcoreMesh(core_axis_name="c", subcore_axis_name="s")
    return pl.kernel(
        _pool_body,
        out_type=jax.ShapeDtypeStruct((B, EMB), jnp.float32),
        mesh=mesh,
        scratch_types=[
            pltpu.VMEM((CHUNKS, ROWS_PER_CHUNK), jnp.int32),
            pltpu.VMEM((ROWS_PER_CHUNK, EMB), jnp.float32),
            pltpu.VMEM((ROWS_PER_CHUNK, EMB), jnp.float32),
            pltpu.VMEM((BPW, EMB), jnp.float32),
            pltpu.SemaphoreType.DMA,
            pltpu.SemaphoreType.DMA,
        ],
        compiler_params=pltpu.CompilerParams(use_tc_tiling_on_sc=False),
    )(idx3, emb_table)


def _mlp_body(x_ref, w1_ref, b1_ref, w2_ref, b2_ref, o_ref):
    x = x_ref[...]
    h = jnp.dot(x, w1_ref[...], preferred_element_type=jnp.float32)
    h = jnp.maximum(h + b1_ref[...], 0.0)
    logits = jnp.dot(h, w2_ref[...], preferred_element_type=jnp.float32)
    logits = logits + b2_ref[...]
    m = jnp.max(logits, axis=-1, keepdims=True)
    e = jnp.exp(logits - m)
    lse = jnp.log(jnp.sum(e, axis=-1, keepdims=True)) + m
    o_ref[...] = logits - lse


def _mlp(pooled, W1, b1, W2, b2):
    BM = 2048
    grid = (B // BM,)
    return pl.pallas_call(
        _mlp_body,
        grid=grid,
        in_specs=[
            pl.BlockSpec((BM, EMB), lambda i: (i, 0)),
            pl.BlockSpec((EMB, HID), lambda i: (0, 0)),
            pl.BlockSpec((1, HID), lambda i: (0, 0)),
            pl.BlockSpec((HID, OUT), lambda i: (0, 0)),
            pl.BlockSpec((1, OUT), lambda i: (0, 0)),
        ],
        out_specs=pl.BlockSpec((BM, OUT), lambda i: (i, 0)),
        out_shape=jax.ShapeDtypeStruct((B, OUT), jnp.float32),
    )(pooled, W1, b1.reshape(1, HID), W2, b2.reshape(1, OUT))


def kernel(input, emb_table, W1, b1, W2, b2):
    idx3 = input.astype(jnp.int32).reshape(NW, CHUNKS, ROWS_PER_CHUNK)
    pooled = _gather_pool(idx3, emb_table)
    return _mlp(pooled, W1, b1, W2, b2)
